# Initial kernel scaffold; baseline (speedup 1.0000x reference)
import jax
import jax.numpy as jnp
from jax.experimental import pallas as pl


def kernel(student_output):
    raise NotImplementedError("write your pallas kernel here")


# speedup vs baseline: 37.4769x; 37.4769x over previous
import jax
import jax.numpy as jnp
from jax.experimental import pallas as pl
from jax.experimental.pallas import tpu as pltpu

_EPS = 1e-8
_B = 4096
_D = 256
_BLK = 512
_R = _B // _BLK


def _koleo_body(x_ref, acc_ref, xn_ref, s_ref, q_ref):
    i = pl.program_id(0)

    @pl.when(i == 0)
    def _init():
        x = x_ref[...]
        nrm = jnp.sqrt(jnp.sum(x * x, axis=1, keepdims=True))
        xn = x / jnp.maximum(nrm, _EPS)
        xn_ref[...] = xn
        ones = jnp.ones((1, _D), jnp.float32)
        s_ref[...] = jax.lax.dot_general(
            ones, xn, (((1,), (1,)), ((), ())),
            preferred_element_type=jnp.float32)
        q_ref[...] = jax.lax.dot_general(
            ones, xn * xn, (((1,), (1,)), ((), ())),
            preferred_element_type=jnp.float32)

    xn = xn_ref[...]
    xi = xn_ref[pl.ds(i * _BLK, _BLK), :]
    dots = jax.lax.dot_general(
        xi, xn, (((1,), (1,)), ((), ())),
        preferred_element_type=jnp.float32)
    rows = i * _BLK + jax.lax.broadcasted_iota(jnp.int32, (_BLK, _B), 0)
    cols = jax.lax.broadcasted_iota(jnp.int32, (_BLK, _B), 1)
    dots = jnp.where(rows == cols, jnp.float32(-1.0), dots)

    m = jnp.max(dots, axis=1, keepdims=True)
    hit = dots >= m
    neg = jnp.float32(-3e38)
    s_bc = jnp.broadcast_to(s_ref[...], (_BLK, _B))
    q_bc = jnp.broadcast_to(q_ref[...], (_BLK, _B))
    ssel = jnp.max(jnp.where(hit, s_bc, neg), axis=1, keepdims=True)
    qsel = jnp.max(jnp.where(hit, q_bc, neg), axis=1, keepdims=True)

    si = jnp.sum(xi, axis=1, keepdims=True)
    qi = jnp.sum(xi * xi, axis=1, keepdims=True)
    dist2 = (qi + qsel - 2.0 * m
             + 2.0 * _EPS * (si - ssel) + _D * _EPS * _EPS)
    dist = jnp.sqrt(jnp.maximum(dist2, 0.0))
    part = jnp.sum(jnp.log(dist + _EPS))

    @pl.when(i == 0)
    def _first():
        acc_ref[0, 0] = part

    @pl.when(i > 0)
    def _rest():
        acc_ref[0, 0] = acc_ref[0, 0] + part


def kernel(student_output):
    acc = pl.pallas_call(
        _koleo_body,
        grid=(_R,),
        in_specs=[pl.BlockSpec((_B, _D), lambda i: (0, 0))],
        out_specs=pl.BlockSpec(
            block_shape=(1, 1),
            index_map=lambda i: (0, 0),
            memory_space=pltpu.SMEM,
        ),
        out_shape=jax.ShapeDtypeStruct((1, 1), jnp.float32),
        scratch_shapes=[
            pltpu.VMEM((_B, _D), jnp.float32),
            pltpu.VMEM((1, _B), jnp.float32),
            pltpu.VMEM((1, _B), jnp.float32),
        ],
        compiler_params=pltpu.CompilerParams(
            dimension_semantics=("arbitrary",)),
    )(student_output)
    return -(acc[0, 0] / _B)


# speedup vs baseline: 45.2475x; 1.2073x over previous
import jax
import jax.numpy as jnp
from jax.experimental import pallas as pl
from jax.experimental.pallas import tpu as pltpu

_EPS = 1e-8
_B = 4096
_D = 256
_BLK = 512
_R = _B // _BLK
_CH = 512
_NCH = _B // _CH


def _koleo_body(x_ref, acc_ref, xn_ref, c_ref):
    i = pl.program_id(0)

    @pl.when(i == 0)
    def _init():
        x = x_ref[...]
        nrm = jnp.sqrt(jnp.sum(x * x, axis=1, keepdims=True))
        xn = x / jnp.maximum(nrm, _EPS)
        xn_ref[...] = xn
        ones = jnp.ones((1, _D), jnp.float32)
        s_row = jax.lax.dot_general(
            ones, xn, (((1,), (1,)), ((), ())),
            preferred_element_type=jnp.float32)
        q_row = jax.lax.dot_general(
            ones, xn * xn, (((1,), (1,)), ((), ())),
            preferred_element_type=jnp.float32)
        c_ref[...] = q_row - (2.0 * _EPS) * s_row

    xi = xn_ref[pl.ds(i * _BLK, _BLK), :]
    rows = i * _BLK + jax.lax.broadcasted_iota(jnp.int32, (_BLK, _CH), 0)
    neg = jnp.float32(-3e38)

    m = jnp.full((_BLK, 1), neg, jnp.float32)
    csel = jnp.full((_BLK, 1), neg, jnp.float32)
    for k in range(_NCH):
        xc = xn_ref[pl.ds(k * _CH, _CH), :]
        dch = jax.lax.dot_general(
            xi, xc, (((1,), (1,)), ((), ())),
            preferred_element_type=jnp.float32)
        cols = k * _CH + jax.lax.broadcasted_iota(jnp.int32, (_BLK, _CH), 1)
        dch = jnp.where(rows == cols, jnp.float32(-1.0), dch)
        cm = jnp.max(dch, axis=1, keepdims=True)
        cc = jnp.max(
            jnp.where(dch >= cm, c_ref[0:1, pl.ds(k * _CH, _CH)], neg),
            axis=1, keepdims=True)
        better = cm > m
        csel = jnp.where(better, cc, csel)
        m = jnp.maximum(cm, m)

    si = jnp.sum(xi, axis=1, keepdims=True)
    qi = jnp.sum(xi * xi, axis=1, keepdims=True)
    dist2 = qi + (2.0 * _EPS) * si - 2.0 * m + csel + _D * _EPS * _EPS
    dist = jnp.sqrt(jnp.maximum(dist2, 0.0))
    part = jnp.sum(jnp.log(dist + _EPS))

    @pl.when(i == 0)
    def _first():
        acc_ref[0, 0] = part

    @pl.when(i > 0)
    def _rest():
        acc_ref[0, 0] = acc_ref[0, 0] + part


def kernel(student_output):
    acc = pl.pallas_call(
        _koleo_body,
        grid=(_R,),
        in_specs=[pl.BlockSpec((_B, _D), lambda i: (0, 0))],
        out_specs=pl.BlockSpec(
            block_shape=(1, 1),
            index_map=lambda i: (0, 0),
            memory_space=pltpu.SMEM,
        ),
        out_shape=jax.ShapeDtypeStruct((1, 1), jnp.float32),
        scratch_shapes=[
            pltpu.VMEM((_B, _D), jnp.float32),
            pltpu.VMEM((1, _B), jnp.float32),
        ],
        compiler_params=pltpu.CompilerParams(
            dimension_semantics=("arbitrary",)),
    )(student_output)
    return -(acc[0, 0] / _B)
